# Initial kernel scaffold; baseline (speedup 1.0000x reference)
import jax
import jax.numpy as jnp
from jax.experimental import pallas as pl


def kernel(h, edge_index):
    raise NotImplementedError("write your pallas kernel here")


# speedup vs baseline: 2.0999x; 2.0999x over previous
import functools

import jax
import jax.numpy as jnp
from jax import lax
from jax.experimental import pallas as pl
from jax.experimental.pallas import tpu as pltpu
from jax.experimental.pallas import tpu_sc as plsc

_LANES = 16


def _sc_body(n_chunks, chunk, d_feat, h_hbm, src_hbm, dst_hbm, out_hbm,
             idx_u, idx_v, rows_u, rows_v, out_buf, sem_u, sem_v):
    n_cores = 2
    wid = lax.axis_index("s") * n_cores + lax.axis_index("c")
    w_base = wid * (n_chunks * chunk)

    def step(g, carry):
        base = pl.multiple_of(w_base + g * chunk, 8)
        pltpu.sync_copy(src_hbm.at[pl.ds(base, chunk)], idx_u)
        pltpu.sync_copy(dst_hbm.at[pl.ds(base, chunk)], idx_v)
        cp_u = pltpu.async_copy(h_hbm.at[idx_u], rows_u, sem_u)
        cp_v = pltpu.async_copy(h_hbm.at[idx_v], rows_v, sem_v)
        cp_u.wait()
        cp_v.wait()
        lane = lax.iota(jnp.int32, _LANES)
        for eb in range(chunk // _LANES):
            scores = jnp.zeros((_LANES,), jnp.float32)
            for el in range(_LANES):
                e = eb * _LANES + el
                p = jnp.zeros((_LANES,), jnp.float32)
                for db in range(d_feat // _LANES):
                    u = rows_u[e, pl.ds(db * _LANES, _LANES)]
                    v = rows_v[e, pl.ds(db * _LANES, _LANES)]
                    p = p + u * v
                s = jnp.sum(p)
                scores = jnp.where(lane == el, s, scores)
            out_buf[pl.ds(eb * _LANES, _LANES)] = scores
        pltpu.sync_copy(out_buf, out_hbm.at[pl.ds(base, chunk)])
        return carry

    lax.fori_loop(0, n_chunks, step, 0)


def kernel(h, edge_index):
    n_nodes, d_feat = h.shape
    n_edges = edge_index.shape[1]
    n_workers = 32
    chunk = 80
    assert n_edges % (n_workers * chunk) == 0
    n_chunks = n_edges // (n_workers * chunk)

    src = edge_index[0]
    dst = edge_index[1]

    mesh = plsc.VectorSubcoreMesh(core_axis_name="c", subcore_axis_name="s")
    body = functools.partial(_sc_body, n_chunks, chunk, d_feat)
    run = pl.kernel(
        body,
        mesh=mesh,
        compiler_params=pltpu.CompilerParams(needs_layout_passes=False),
        out_type=jax.ShapeDtypeStruct((n_edges,), jnp.float32),
        scratch_types=[
            pltpu.VMEM((chunk,), jnp.int32),
            pltpu.VMEM((chunk,), jnp.int32),
            pltpu.VMEM((chunk, d_feat), jnp.float32),
            pltpu.VMEM((chunk, d_feat), jnp.float32),
            pltpu.VMEM((chunk,), jnp.float32),
            pltpu.SemaphoreType.DMA,
            pltpu.SemaphoreType.DMA,
        ],
    )
    return run(h, src, dst)


# speedup vs baseline: 4.1772x; 1.9893x over previous
import functools

import jax
import jax.numpy as jnp
from jax import lax
from jax.experimental import pallas as pl
from jax.experimental.pallas import tpu as pltpu
from jax.experimental.pallas import tpu_sc as plsc

_LANES = 16
_WORKERS = 32
_CHUNK = 80


def _sc_body(n_chunks, chunk, d_feat, h_hbm, src_hbm, dst_hbm, out_hbm,
             idx_u, idx_v, ru0, rv0, ru1, rv1, out_l,
             sem0, sem1):
    n_cores = 2
    wid = lax.axis_index("s") * n_cores + lax.axis_index("c")

    pltpu.sync_copy(src_hbm.at[wid], idx_u)
    pltpu.sync_copy(dst_hbm.at[wid], idx_v)

    def start(g, ru, rv, sem):
        cu = pltpu.async_copy(h_hbm.at[idx_u.at[g]], ru, sem)
        cv = pltpu.async_copy(h_hbm.at[idx_v.at[g]], rv, sem)
        return cu, cv

    def wait(ru, rv, sem):
        pltpu.make_async_copy(h_hbm.at[idx_u.at[0]], ru, sem).wait()
        pltpu.make_async_copy(h_hbm.at[idx_v.at[0]], rv, sem).wait()

    lane = lax.iota(jnp.int32, _LANES)

    def compute(g, ru, rv):
        for eb in range(chunk // _LANES):
            scores = jnp.zeros((_LANES,), jnp.float32)
            for el in range(_LANES):
                e = eb * _LANES + el
                p = jnp.zeros((_LANES,), jnp.float32)
                for db in range(d_feat // _LANES):
                    u = ru[e, pl.ds(db * _LANES, _LANES)]
                    v = rv[e, pl.ds(db * _LANES, _LANES)]
                    p = p + u * v
                s = jnp.sum(p)
                scores = jnp.where(lane == el, s, scores)
            out_l[g, pl.ds(eb * _LANES, _LANES)] = scores

    start(0, ru0, rv0, sem0)

    def body2(i, carry):
        g0 = i * 2
        start(g0 + 1, ru1, rv1, sem1)
        wait(ru0, rv0, sem0)
        compute(g0, ru0, rv0)
        start(g0 + 2, ru0, rv0, sem0)
        wait(ru1, rv1, sem1)
        compute(g0 + 1, ru1, rv1)
        return carry

    lax.fori_loop(0, (n_chunks - 1) // 2, body2, 0)
    wait(ru0, rv0, sem0)
    compute(n_chunks - 1, ru0, rv0)

    pltpu.sync_copy(out_l, out_hbm.at[wid])


def kernel(h, edge_index):
    n_nodes, d_feat = h.shape
    n_edges = edge_index.shape[1]
    assert n_edges % (_WORKERS * _CHUNK) == 0
    n_chunks = n_edges // (_WORKERS * _CHUNK)

    src = edge_index[0].reshape(_WORKERS, n_chunks, _CHUNK)
    dst = edge_index[1].reshape(_WORKERS, n_chunks, _CHUNK)

    mesh = plsc.VectorSubcoreMesh(core_axis_name="c", subcore_axis_name="s")
    body = functools.partial(_sc_body, n_chunks, _CHUNK, d_feat)
    run = pl.kernel(
        body,
        mesh=mesh,
        compiler_params=pltpu.CompilerParams(needs_layout_passes=False),
        out_type=jax.ShapeDtypeStruct((_WORKERS, n_chunks, _CHUNK),
                                      jnp.float32),
        scratch_types=[
            pltpu.VMEM((n_chunks, _CHUNK), jnp.int32),
            pltpu.VMEM((n_chunks, _CHUNK), jnp.int32),
            pltpu.VMEM((_CHUNK, d_feat), jnp.float32),
            pltpu.VMEM((_CHUNK, d_feat), jnp.float32),
            pltpu.VMEM((_CHUNK, d_feat), jnp.float32),
            pltpu.VMEM((_CHUNK, d_feat), jnp.float32),
            pltpu.VMEM((n_chunks, _CHUNK), jnp.float32),
            pltpu.SemaphoreType.DMA,
            pltpu.SemaphoreType.DMA,
        ],
    )
    return run(h, src, dst).reshape(n_edges)
